# baseline (speedup 1.0000x reference)
import functools

import jax
import jax.numpy as jnp
from jax import lax
from jax.experimental import pallas as pl

BATCH = 512
NUM_TTA = 64
NUM_CLASSES = 1000
KEPT_VIEWS = 6
EPS = 1e-08

ROWS_A = 256
GRID_A = (BATCH * NUM_TTA) // ROWS_A
SAMP_B = 8
GRID_B = BATCH // SAMP_B


def _stage_a(x_ref, ent_ref, vote_ref):
    p = x_ref[...]
    safe = jnp.where(p > 0, p, 1.0)
    e = jnp.where(p > 0, -p * jnp.log(safe),
                  jnp.where(p == 0, 0.0, -jnp.inf))
    ent_ref[0, 0, :] = jnp.sum(e, axis=1)
    m = jnp.max(p, axis=1)
    lane = lax.broadcasted_iota(jnp.int32, p.shape, 1)
    idx = jnp.min(jnp.where(p == m[:, None], lane, NUM_CLASSES), axis=1)
    vote_ref[0, 0, :] = idx


def _stage_b(ent_ref, vote_ref, out_ref):
    S, V = SAMP_B, NUM_TTA
    i32 = jnp.int32
    e = ent_ref[...]
    v = vote_ref[...]

    ii = lax.broadcasted_iota(i32, (S, V, V), 1)
    jj = lax.broadcasted_iota(i32, (S, V, V), 2)

    ei = e[:, :, None]
    ej = e[:, None, :]
    before = (ej < ei) | ((ej == ei) & (jj < ii))
    rank = jnp.sum(before.astype(i32), axis=2)

    onehot = rank[:, None, :] == ii
    sv = jnp.sum(jnp.where(onehot, v[:, None, :], 0), axis=2)

    svi = sv[:, :, None]
    svj = sv[:, None, :]
    occ = jnp.sum(((svj == svi) & (jj <= ii)).astype(i32), axis=2)

    occj = occ[:, None, :]
    kmask = jj <= ii
    mx = jnp.max(jnp.where(kmask, occj, 0), axis=2)
    nmx = jnp.sum((kmask & (occj == mx[:, :, None])).astype(i32), axis=2)

    kk = lax.broadcasted_iota(i32, (S, V), 1)
    k = kk + 1
    ok = (k >= KEPT_VIEWS) & (nmx <= 1)
    ksel = jnp.min(jnp.where(ok, k, V), axis=1, keepdims=True)

    lanes = lax.broadcasted_iota(i32, (S, V), 1)
    cls = lax.broadcasted_iota(i32, (S, NUM_CLASSES), 1)

    def body(i, acc):
        col = jnp.sum(jnp.where(lanes == i, sv, 0), axis=1, keepdims=True)
        act = i < ksel
        hit = (cls == col) & act
        return acc + hit.astype(jnp.float32)

    counts = lax.fori_loop(0, V, body,
                           jnp.zeros((S, NUM_CLASSES), jnp.float32))
    out_ref[...] = jnp.log(counts * (1.0 / NUM_TTA) + EPS)


@jax.jit
def kernel(x):
    total = BATCH * NUM_TTA
    ent3, vote3 = pl.pallas_call(
        _stage_a,
        grid=(GRID_A,),
        in_specs=[pl.BlockSpec((ROWS_A, NUM_CLASSES), lambda i: (i, 0))],
        out_specs=[
            pl.BlockSpec((1, 1, ROWS_A), lambda i: (i, 0, 0)),
            pl.BlockSpec((1, 1, ROWS_A), lambda i: (i, 0, 0)),
        ],
        out_shape=[
            jax.ShapeDtypeStruct((GRID_A, 1, ROWS_A), jnp.float32),
            jax.ShapeDtypeStruct((GRID_A, 1, ROWS_A), jnp.int32),
        ],
    )(x)
    ent = ent3.reshape(BATCH, NUM_TTA)
    votes = vote3.reshape(BATCH, NUM_TTA)

    out = pl.pallas_call(
        _stage_b,
        grid=(GRID_B,),
        in_specs=[
            pl.BlockSpec((SAMP_B, NUM_TTA), lambda b: (b, 0)),
            pl.BlockSpec((SAMP_B, NUM_TTA), lambda b: (b, 0)),
        ],
        out_specs=pl.BlockSpec((SAMP_B, NUM_CLASSES), lambda b: (b, 0)),
        out_shape=jax.ShapeDtypeStruct((BATCH, NUM_CLASSES), jnp.float32),
    )(ent, votes)
    return out


# speedup vs baseline: 5.7498x; 5.7498x over previous
import functools

import jax
import jax.numpy as jnp
from jax import lax
from jax.experimental import pallas as pl
from jax.experimental.pallas import tpu as pltpu
from jax.experimental.pallas import tpu_sc as plsc

BATCH = 512
NUM_TTA = 64
NUM_CLASSES = 1000
KEPT_VIEWS = 6
EPS = 1e-08

ROWS_A = 256
GRID_A = (BATCH * NUM_TTA) // ROWS_A

NC = 2
NS = 16
L = 16
NW = NC * NS
SAMP_SC = BATCH // NW
TAB = 80


def _stage_a(x_ref, ent_ref, vote_ref, tab_ref):
    p = x_ref[...]
    safe = jnp.maximum(p, jnp.float32(1e-37))
    ent_ref[0, 0, :] = -jnp.sum(p * jnp.log(safe), axis=1)
    m = jnp.max(p, axis=1)
    lane = lax.broadcasted_iota(jnp.int32, p.shape, 1)
    idx = jnp.min(jnp.where(p == m[:, None], lane, NUM_CLASSES), axis=1)
    vote_ref[0, 0, :] = idx

    @pl.when(pl.program_id(0) == 0)
    def _():
        c = lax.broadcasted_iota(jnp.int32, (1, 128), 1).astype(jnp.float32)
        tab_ref[...] = jnp.log(c * (1.0 / NUM_TTA) + EPS)


def _sc_body(entW, votesW, logtab, out,
             ent_v, votes_v, sv_f, cnt_f, rows_v, tab_v):
    i32 = jnp.int32
    wid = lax.axis_index("s") * NC + lax.axis_index("c")
    base = wid * SAMP_SC
    pltpu.sync_copy(entW.at[wid], ent_v)
    pltpu.sync_copy(votesW.at[wid], votes_v)
    pltpu.sync_copy(logtab, tab_v)

    lane = lax.iota(i32, L)
    zeros = jnp.zeros((L,), i32)

    def zbody(r, _):
        cnt_f[pl.ds(r * L, L)] = zeros
        return 0
    lax.fori_loop(0, NUM_CLASSES, zbody, 0)

    def rbody(i, _):
        e_i = ent_v[i, :]

        def jlo(j, r):
            return r + jnp.where(ent_v[j, :] <= e_i, 1, 0)

        def jhi(j, r):
            return r + jnp.where(ent_v[j, :] < e_i, 1, 0)

        r = lax.fori_loop(0, i, jlo, zeros)
        r = lax.fori_loop(i + 1, NUM_TTA, jhi, r)
        plsc.store_scatter(sv_f, [r * L + lane], votes_v[i, :])
        return 0
    lax.fori_loop(0, NUM_TTA, rbody, 0)

    def vbody(t, carry):
        mx, nmx, active = carry
        v_t = sv_f[pl.ds(t * L, L)]
        idx = v_t * L + lane
        new = plsc.load_gather(cnt_f, [idx]) + 1
        plsc.store_scatter(cnt_f, [idx], new, mask=active)
        upd = active & (new > mx)
        tie = active & (new == mx)
        mx = jnp.where(upd, new, mx)
        nmx = jnp.where(upd, 1, jnp.where(tie, nmx + 1, nmx))
        stop = (t + 1 >= KEPT_VIEWS) & (nmx == 1)
        active = active & jnp.logical_not(stop)
        return mx, nmx, active
    lax.fori_loop(0, NUM_TTA, vbody,
                  (zeros, zeros, jnp.ones((L,), jnp.bool_)))

    def srow(s, _):
        def cchunk(k, _):
            c0 = jnp.minimum(k * L, NUM_CLASSES - L)
            cnts = plsc.load_gather(cnt_f, [(lane + c0) * L + s])
            rows_v[s, pl.ds(c0, L)] = plsc.load_gather(tab_v, [cnts])
            return 0
        lax.fori_loop(0, (NUM_CLASSES + L - 1) // L, cchunk, 0)
        return 0
    lax.fori_loop(0, SAMP_SC, srow, 0)
    pltpu.sync_copy(rows_v, out.at[pl.ds(base, SAMP_SC)])


_stage_b_sc = functools.partial(
    pl.kernel,
    mesh=plsc.VectorSubcoreMesh(core_axis_name="c", subcore_axis_name="s"),
    compiler_params=pltpu.CompilerParams(needs_layout_passes=False),
    out_type=jax.ShapeDtypeStruct((BATCH, NUM_CLASSES), jnp.float32),
    scratch_types=[
        pltpu.VMEM((NUM_TTA, SAMP_SC), jnp.float32),
        pltpu.VMEM((NUM_TTA, SAMP_SC), jnp.int32),
        pltpu.VMEM((NUM_TTA * L,), jnp.int32),
        pltpu.VMEM((NUM_CLASSES * L,), jnp.int32),
        pltpu.VMEM((SAMP_SC, NUM_CLASSES), jnp.float32),
        pltpu.VMEM((TAB,), jnp.float32),
    ],
)(_sc_body)


@jax.jit
def kernel(x):
    ent3, vote3, tab = pl.pallas_call(
        _stage_a,
        grid=(GRID_A,),
        in_specs=[pl.BlockSpec((ROWS_A, NUM_CLASSES), lambda i: (i, 0))],
        out_specs=[
            pl.BlockSpec((1, 1, ROWS_A), lambda i: (i, 0, 0)),
            pl.BlockSpec((1, 1, ROWS_A), lambda i: (i, 0, 0)),
            pl.BlockSpec((1, 128), lambda i: (0, 0)),
        ],
        out_shape=[
            jax.ShapeDtypeStruct((GRID_A, 1, ROWS_A), jnp.float32),
            jax.ShapeDtypeStruct((GRID_A, 1, ROWS_A), jnp.int32),
            jax.ShapeDtypeStruct((1, 128), jnp.float32),
        ],
    )(x)
    entW = ent3.reshape(NW, SAMP_SC, NUM_TTA).transpose(0, 2, 1)
    votesW = vote3.reshape(NW, SAMP_SC, NUM_TTA).transpose(0, 2, 1)
    logtab = tab.reshape(128)[:TAB]
    return _stage_b_sc(entW, votesW, logtab)


# speedup vs baseline: 8.0259x; 1.3959x over previous
import functools

import jax
import jax.numpy as jnp
from jax import lax
from jax.experimental import pallas as pl
from jax.experimental.pallas import tpu as pltpu
from jax.experimental.pallas import tpu_sc as plsc

BATCH = 512
NUM_TTA = 64
NUM_CLASSES = 1000
KEPT_VIEWS = 6
EPS = 1e-08

SAMP_A = 8
GRID_A = BATCH // SAMP_A

NC = 2
NS = 16
L = 16
NW = NC * NS
SAMP_SC = BATCH // NW
TAB = 80


def _stage_a(x_ref, ent_ref, vote_ref, tab_ref):
    p = x_ref[...]
    safe = jnp.maximum(p, jnp.float32(1e-37))
    ent_ref[...] = -jnp.sum(p * jnp.log(safe), axis=2)
    m = jnp.max(p, axis=2)
    lane = lax.broadcasted_iota(jnp.int32, p.shape, 2)
    vote_ref[...] = jnp.min(
        jnp.where(p == m[:, :, None], lane, NUM_CLASSES), axis=2)

    @pl.when(pl.program_id(0) == 0)
    def _():
        c = lax.broadcasted_iota(jnp.int32, (1, 128), 1).astype(jnp.float32)
        tab_ref[...] = jnp.log(c * (1.0 / NUM_TTA) + EPS)


def _sc_body(ent_sm, votes_sm, logtab, out,
             ent_sv, votes_sv, ent_v, votes_v, rank_v, sv_f, cnt_f,
             rows_v, tab_v):
    i32 = jnp.int32
    wid = lax.axis_index("s") * NC + lax.axis_index("c")
    base = wid * SAMP_SC
    pltpu.sync_copy(ent_sm.at[pl.ds(base, SAMP_SC)], ent_sv)
    pltpu.sync_copy(votes_sm.at[pl.ds(base, SAMP_SC)], votes_sv)
    pltpu.sync_copy(logtab, tab_v)

    lane = lax.iota(i32, L)
    zeros = jnp.zeros((L,), i32)

    def tbody(i, _):
        vcol = jnp.full((L,), i, i32)
        ent_v[i, :] = plsc.load_gather(ent_sv, [lane, vcol])
        votes_v[i, :] = plsc.load_gather(votes_sv, [lane, vcol])
        return 0
    lax.fori_loop(0, NUM_TTA, tbody, 0, unroll=8)

    def zbody(r, _):
        cnt_f[pl.ds(r * L, L)] = zeros
        return 0
    lax.fori_loop(0, NUM_CLASSES, zbody, 0, unroll=8)

    def rbody(ib, ssum):
        e_a = ent_v[2 * ib, :]
        e_b = ent_v[2 * ib + 1, :]

        def jall(j, rs):
            ra, rb = rs
            e_j = ent_v[j, :]
            ra = ra + jnp.where(e_j < e_a, 1, 0)
            rb = rb + jnp.where(e_j < e_b, 1, 0)
            return ra, rb

        ra, rb = lax.fori_loop(0, NUM_TTA, jall, (zeros, zeros), unroll=8)
        rank_v[2 * ib, :] = ra
        rank_v[2 * ib + 1, :] = rb
        return ssum + ra + rb
    ssum = lax.fori_loop(0, NUM_TTA // 2, rbody, zeros)

    tie_any = jnp.max(jnp.where(ssum != NUM_TTA * (NUM_TTA - 1) // 2, 1, 0))

    @pl.when(tie_any > 0)
    def _():
        def fix(i, _):
            e_i = ent_v[i, :]

            def jeq(j, r):
                return r + jnp.where(ent_v[j, :] == e_i, 1, 0)

            extra = lax.fori_loop(0, i, jeq, zeros)
            rank_v[i, :] = rank_v[i, :] + extra
            return 0
        lax.fori_loop(0, NUM_TTA, fix, 0)

    def sbody(i, _):
        plsc.store_scatter(sv_f, [rank_v[i, :] * L + lane], votes_v[i, :])
        return 0
    lax.fori_loop(0, NUM_TTA, sbody, 0, unroll=8)

    def vbody(t, carry):
        mx, nmx, active = carry
        v_t = sv_f[pl.ds(t * L, L)]
        idx = v_t * L + lane
        new = plsc.load_gather(cnt_f, [idx]) + 1
        plsc.store_scatter(cnt_f, [idx], new, mask=active)
        upd = active & (new > mx)
        tie = active & (new == mx)
        mx = jnp.where(upd, new, mx)
        nmx = jnp.where(upd, 1, jnp.where(tie, nmx + 1, nmx))
        stop = (t + 1 >= KEPT_VIEWS) & (nmx == 1)
        active = active & jnp.logical_not(stop)
        return mx, nmx, active
    lax.fori_loop(0, NUM_TTA, vbody,
                  (zeros, zeros, jnp.ones((L,), jnp.bool_)))

    def srow(s, _):
        def cchunk(k, _):
            c0 = jnp.minimum(k * L, NUM_CLASSES - L)
            cnts = plsc.load_gather(cnt_f, [(lane + c0) * L + s])
            rows_v[s, pl.ds(c0, L)] = plsc.load_gather(tab_v, [cnts])
            return 0
        lax.fori_loop(0, (NUM_CLASSES + L - 1) // L, cchunk, 0, unroll=4)
        return 0
    lax.fori_loop(0, SAMP_SC, srow, 0)
    pltpu.sync_copy(rows_v, out.at[pl.ds(base, SAMP_SC)])


_stage_b_sc = functools.partial(
    pl.kernel,
    mesh=plsc.VectorSubcoreMesh(core_axis_name="c", subcore_axis_name="s"),
    compiler_params=pltpu.CompilerParams(needs_layout_passes=False),
    out_type=jax.ShapeDtypeStruct((BATCH, NUM_CLASSES), jnp.float32),
    scratch_types=[
        pltpu.VMEM((SAMP_SC, NUM_TTA), jnp.float32),
        pltpu.VMEM((SAMP_SC, NUM_TTA), jnp.int32),
        pltpu.VMEM((NUM_TTA, SAMP_SC), jnp.float32),
        pltpu.VMEM((NUM_TTA, SAMP_SC), jnp.int32),
        pltpu.VMEM((NUM_TTA, SAMP_SC), jnp.int32),
        pltpu.VMEM((NUM_TTA * L,), jnp.int32),
        pltpu.VMEM((NUM_CLASSES * L,), jnp.int32),
        pltpu.VMEM((SAMP_SC, NUM_CLASSES), jnp.float32),
        pltpu.VMEM((TAB,), jnp.float32),
    ],
)(_sc_body)


@jax.jit
def kernel(x):
    x3 = x.reshape(BATCH, NUM_TTA, NUM_CLASSES)
    ent, votes, tab = pl.pallas_call(
        _stage_a,
        grid=(GRID_A,),
        in_specs=[
            pl.BlockSpec((SAMP_A, NUM_TTA, NUM_CLASSES), lambda i: (i, 0, 0)),
        ],
        out_specs=[
            pl.BlockSpec((SAMP_A, NUM_TTA), lambda i: (i, 0)),
            pl.BlockSpec((SAMP_A, NUM_TTA), lambda i: (i, 0)),
            pl.BlockSpec((1, 128), lambda i: (0, 0)),
        ],
        out_shape=[
            jax.ShapeDtypeStruct((BATCH, NUM_TTA), jnp.float32),
            jax.ShapeDtypeStruct((BATCH, NUM_TTA), jnp.int32),
            jax.ShapeDtypeStruct((1, 128), jnp.float32),
        ],
    )(x3)
    logtab = tab.reshape(128)[:TAB]
    return _stage_b_sc(ent, votes, logtab)
